# Initial kernel scaffold; baseline (speedup 1.0000x reference)
import jax
import jax.numpy as jnp
from jax.experimental import pallas as pl


def kernel(x, edge_index, edge_weight, W1, b1, W2, b2, init_mu, num_iter):
    raise NotImplementedError("write your pallas kernel here")


# speedup vs baseline: 1.7829x; 1.7829x over previous
import functools

import jax
import jax.numpy as jnp
from jax import lax
from jax.experimental import pallas as pl
from jax.experimental.pallas import tpu as pltpu
from jax.experimental.pallas import tpu_sc as plsc

NC = 2
NS = 16
NW = NC * NS
LANES = 16
CHUNK = 128
DSP = 128
CLUSTER_TEMP = 30.0
_F32 = jnp.float32


def _spmm_body(n, nchunks, sup_hbm, src_hbm, dst_hbm, w_hbm, out_hbm,
               acc, src_v, dst_v, w_v, rows_v, zbuf, sem):
    c = lax.axis_index("c")
    s = lax.axis_index("s")
    wid = c * NS + s
    rows_per_tile = n // NS
    zrows = 32

    zero16 = jnp.zeros((LANES,), _F32)
    for i in range(zrows):
        for k in range(DSP // LANES):
            zbuf[i, pl.ds(k * LANES, LANES)] = zero16

    def zloop(i, _):
        pltpu.sync_copy(zbuf, acc.at[pl.ds(s * rows_per_tile + i * zrows, zrows)])
        return 0
    lax.fori_loop(0, rows_per_tile // zrows, zloop, 0)

    pltpu.sync_copy(src_hbm.at[wid], src_v)
    pltpu.sync_copy(dst_hbm.at[wid], dst_v)

    plsc.subcore_barrier()

    def chunk_body(j, _):
        pltpu.sync_copy(w_hbm.at[wid, j], w_v)
        pltpu.async_copy(sup_hbm.at[src_v.at[j]], rows_v, sem).wait()

        def escale(e, _):
            w16 = w_v[e]
            for k in range(DSP // LANES):
                sl = pl.ds(k * LANES, LANES)
                rows_v[e, sl] = rows_v[e, sl] * w16
            return 0
        lax.fori_loop(0, CHUNK, escale, 0)

        pltpu.sync_copy(rows_v, acc.at[dst_v.at[j]], add=True)
        return 0
    lax.fori_loop(0, nchunks, chunk_body, 0)

    plsc.subcore_barrier()

    pltpu.sync_copy(acc.at[pl.ds(s * rows_per_tile, rows_per_tile)],
                    out_hbm.at[pl.ds(c * n + s * rows_per_tile, rows_per_tile)])


def _make_spmm(n, nchunks):
    mesh = plsc.VectorSubcoreMesh(core_axis_name="c", subcore_axis_name="s",
                                  num_cores=NC, num_subcores=NS)
    return pl.kernel(
        functools.partial(_spmm_body, n, nchunks),
        out_type=jax.ShapeDtypeStruct((NC * n, DSP), _F32),
        mesh=mesh,
        scratch_types=[
            pltpu.VMEM_SHARED((n, DSP), _F32),
            pltpu.VMEM((nchunks, CHUNK), jnp.int32),
            pltpu.VMEM((nchunks, CHUNK), jnp.int32),
            pltpu.VMEM((CHUNK, LANES), _F32),
            pltpu.VMEM((CHUNK, DSP), _F32),
            pltpu.VMEM((32, DSP), _F32),
            pltpu.SemaphoreType.DMA,
        ],
    )


def _mm1_body(x_ref, w_ref, out_ref):
    h = lax.dot_general(x_ref[...], w_ref[...], (((1,), (0,)), ((), ())),
                        preferred_element_type=_F32)
    out_ref[0] = h[:, :DSP]
    out_ref[1] = h[:, DSP:]


def _mm2_body(pa_ref, pb_ref, b1_ref, w2_ref, out_ref):
    h_lo = jnp.maximum(pa_ref[0] + pa_ref[1] + b1_ref[0:1, :], 0.0)
    h_hi = jnp.maximum(pb_ref[0] + pb_ref[1] + b1_ref[1:2, :], 0.0)
    w2 = w2_ref[...]
    out_ref[...] = (
        lax.dot_general(h_lo, w2[:DSP], (((1,), (0,)), ((), ())),
                        preferred_element_type=_F32)
        + lax.dot_general(h_hi, w2[DSP:], (((1,), (0,)), ((), ())),
                          preferred_element_type=_F32))


def _softmax(z):
    m = jnp.max(z, axis=1, keepdims=True)
    e = jnp.exp(z - m)
    return e / jnp.sum(e, axis=1, keepdims=True)


def _emb_body(q_ref, b2_ref, emb_ref, data_ref):
    emb = q_ref[0] + q_ref[1] + b2_ref[0:1, :]
    emb_ref[...] = emb
    nrm = jnp.sqrt(jnp.sum(emb * emb, axis=1, keepdims=True))
    data_ref[...] = emb / nrm


def _cluster_body(data_ref, mu0_ref, niter_ref, mu_ref):
    data = data_ref[...]

    def body(_, mu):
        dist = lax.dot_general(data, mu, (((1,), (1,)), ((), ())),
                               preferred_element_type=_F32)
        r = _softmax(CLUSTER_TEMP * dist)
        cr = jnp.sum(r, axis=0)
        cm = lax.dot_general(r, data, (((0,), (0,)), ((), ())),
                             preferred_element_type=_F32)
        return cm / cr[:, None]

    mu_ref[...] = lax.fori_loop(0, niter_ref[0, 0] + 1, body, mu0_ref[...])


def _assign_body(data_ref, mu_ref, r_ref, dist_ref):
    dist = lax.dot_general(data_ref[...], mu_ref[...], (((1,), (1,)), ((), ())),
                           preferred_element_type=_F32)
    dist_ref[...] = dist
    r_ref[...] = _softmax(CLUSTER_TEMP * dist)


def kernel(x, edge_index, edge_weight, W1, b1, W2, b2, init_mu, num_iter):
    n, nfeat = x.shape
    nout = W2.shape[1]
    k = init_mu.shape[0]
    e = edge_weight.shape[0]

    ep = ((e + NW * CHUNK - 1) // (NW * CHUNK)) * (NW * CHUNK)
    nchunks = ep // (NW * CHUNK)
    pad = ep - e
    dst = jnp.pad(edge_index[0], (0, pad))
    src = jnp.pad(edge_index[1], (0, pad))
    w = jnp.pad(edge_weight, (0, pad))
    src_t = src.reshape(NW, nchunks, CHUNK)
    dst_t = dst.reshape(NW, nchunks, CHUNK)
    w_t = jnp.broadcast_to(w[:, None], (ep, LANES)).reshape(NW, nchunks, CHUNK, LANES)

    n_pad = ((n + 32 * NS - 1) // (32 * NS)) * (32 * NS)
    spmm = _make_spmm(n_pad, nchunks)

    def run_spmm(sup):
        out = spmm(sup, src_t, dst_t, w_t).reshape(2, n_pad, DSP)
        return out[:, :n]

    rb = 1000
    grid = n // rb
    sup1 = pl.pallas_call(
        _mm1_body,
        grid=(grid,),
        in_specs=[pl.BlockSpec((rb, nfeat), lambda i: (i, 0)),
                  pl.BlockSpec((nfeat, 2 * DSP), lambda i: (0, 0))],
        out_specs=pl.BlockSpec((2, rb, DSP), lambda i: (0, i, 0)),
        out_shape=jax.ShapeDtypeStruct((2, n, DSP), _F32),
    )(x, W1)

    pa = run_spmm(sup1[0])
    pb = run_spmm(sup1[1])

    sup2 = pl.pallas_call(
        _mm2_body,
        grid=(grid,),
        in_specs=[pl.BlockSpec((2, rb, DSP), lambda i: (0, i, 0)),
                  pl.BlockSpec((2, rb, DSP), lambda i: (0, i, 0)),
                  pl.BlockSpec((2, DSP), lambda i: (0, 0)),
                  pl.BlockSpec((nfeat, nout), lambda i: (0, 0))],
        out_specs=pl.BlockSpec((rb, nout), lambda i: (i, 0)),
        out_shape=jax.ShapeDtypeStruct((n, nout), _F32),
    )(pa, pb, b1.reshape(2, DSP), W2)

    q = run_spmm(sup2)

    embeds, data = pl.pallas_call(
        _emb_body,
        grid=(grid,),
        in_specs=[pl.BlockSpec((2, rb, DSP), lambda i: (0, i, 0)),
                  pl.BlockSpec((1, nout), lambda i: (0, 0))],
        out_specs=(pl.BlockSpec((rb, nout), lambda i: (i, 0)),
                   pl.BlockSpec((rb, nout), lambda i: (i, 0))),
        out_shape=(jax.ShapeDtypeStruct((n, nout), _F32),
                   jax.ShapeDtypeStruct((n, nout), _F32)),
    )(q, b2.reshape(1, nout))

    niter = jnp.asarray(num_iter, jnp.int32).reshape(1, 1)
    mu = pl.pallas_call(
        _cluster_body,
        in_specs=[pl.BlockSpec(memory_space=pltpu.VMEM),
                  pl.BlockSpec(memory_space=pltpu.VMEM),
                  pl.BlockSpec(memory_space=pltpu.SMEM)],
        out_shape=jax.ShapeDtypeStruct((k, nout), _F32),
    )(data, init_mu, niter)

    r, dist = pl.pallas_call(
        _assign_body,
        grid=(grid,),
        in_specs=[pl.BlockSpec((rb, nout), lambda i: (i, 0)),
                  pl.BlockSpec((k, nout), lambda i: (0, 0))],
        out_specs=(pl.BlockSpec((rb, k), lambda i: (i, 0)),
                   pl.BlockSpec((rb, k), lambda i: (i, 0))),
        out_shape=(jax.ShapeDtypeStruct((n, k), _F32),
                   jax.ShapeDtypeStruct((n, k), _F32)),
    )(data, mu)

    return (mu, r, embeds, dist)


# speedup vs baseline: 2.3801x; 1.3350x over previous
import functools

import jax
import jax.numpy as jnp
from jax import lax
from jax.experimental import pallas as pl
from jax.experimental.pallas import tpu as pltpu
from jax.experimental.pallas import tpu_sc as plsc

NC = 2
NS = 16
NW = NC * NS
LANES = 16
CHUNK = 32
DSP = 128
CLUSTER_TEMP = 30.0
_F32 = jnp.float32


NBUF = 4


def _spmm_body(n, nrow, sup_hbm, src_hbm, dst_hbm, w_hbm, out_hbm,
               acc, src_v, dst_v, w_vs, rows_vs, zbuf,
               g0, g1, g2, g3, s0, s1, s2, s3, zsem):
    gsem = (g0, g1, g2, g3)
    ssem = (s0, s1, s2, s3)
    c = lax.axis_index("c")
    s = lax.axis_index("s")
    wid = c * NS + s
    rpt = n // NS
    zrows = 8
    nz = rpt // zrows
    ng = nrow * 2

    zero16 = jnp.zeros((LANES,), _F32)
    for i in range(zrows):
        for k in range(DSP // LANES):
            zbuf[i, pl.ds(k * LANES, LANES)] = zero16

    def zfire(i, _):
        pltpu.async_copy(zbuf, acc.at[pl.ds(s * rpt + i * zrows, zrows)], zsem)
        return 0
    lax.fori_loop(0, nz, zfire, 0)

    pltpu.sync_copy(src_hbm.at[wid], src_v)
    pltpu.sync_copy(dst_hbm.at[wid], dst_v)

    def zdrain(i, _):
        pltpu.make_async_copy(zbuf, acc.at[pl.ds(0, zrows)], zsem).wait()
        return 0
    lax.fori_loop(0, nz, zdrain, 0)

    for j0 in range(2):
        pltpu.async_copy(w_hbm.at[wid, j0], w_vs.at[j0], gsem[j0])
        pltpu.async_copy(sup_hbm.at[src_v.at[0, pl.ds(j0 * 64, 64)]],
                         rows_vs.at[j0], gsem[j0])

    plsc.subcore_barrier()

    def outer(g, _):
        for b in range(NBUF):
            j = g * NBUF + b
            jr = j // 2
            col0 = (b % 2) * 64
            pltpu.make_async_copy(sup_hbm.at[pl.ds(0, 64)], rows_vs.at[b],
                                  gsem[b]).wait()
            pltpu.make_async_copy(w_hbm.at[wid, 0], w_vs.at[b], gsem[b]).wait()

            def escale(e2, _):
                w16 = w_vs[b, e2 // 8, pl.ds((e2 % 8) * LANES, LANES)]
                for k in range(DSP // LANES):
                    sl = pl.ds(k * LANES, LANES)
                    rows_vs[b, e2, sl] = rows_vs[b, e2, sl] * w16
                return 0
            lax.fori_loop(0, 64, escale, 0)

            pltpu.async_copy(rows_vs.at[b],
                             acc.at[dst_v.at[jr, pl.ds(col0, 64)]],
                             ssem[b], add=True)

            jn = j + 2
            bn = (b + 2) % NBUF

            @pl.when(jn < ng)
            def _():
                @pl.when(j >= 2)
                def _():
                    pltpu.make_async_copy(rows_vs.at[bn], acc.at[pl.ds(0, 64)],
                                          ssem[bn]).wait()
                pltpu.async_copy(w_hbm.at[wid, jn], w_vs.at[bn], gsem[bn])
                pltpu.async_copy(
                    sup_hbm.at[src_v.at[jn // 2, pl.ds(col0, 64)]],
                    rows_vs.at[bn], gsem[bn])
        return 0
    lax.fori_loop(0, ng // NBUF, outer, 0)

    for b in range(NBUF):
        pltpu.make_async_copy(rows_vs.at[b], acc.at[pl.ds(0, 64)],
                              ssem[b]).wait()

    plsc.subcore_barrier()

    pltpu.sync_copy(acc.at[pl.ds(s * rpt, rpt)],
                    out_hbm.at[pl.ds(c * n + s * rpt, rpt)])


def _make_spmm(n, nrow):
    mesh = plsc.VectorSubcoreMesh(core_axis_name="c", subcore_axis_name="s",
                                  num_cores=NC, num_subcores=NS)
    return pl.kernel(
        functools.partial(_spmm_body, n, nrow),
        out_type=jax.ShapeDtypeStruct((NC * n, DSP), _F32),
        mesh=mesh,
        scratch_types=[
            pltpu.VMEM_SHARED((n, DSP), _F32),
            pltpu.VMEM((nrow, 128), jnp.int32),
            pltpu.VMEM((nrow, 128), jnp.int32),
            pltpu.VMEM((NBUF, 8, 128), _F32),
            pltpu.VMEM((NBUF, 64, DSP), _F32),
            pltpu.VMEM((8, DSP), _F32),
        ] + [pltpu.SemaphoreType.DMA] * 9,
    )


def _mm1_body(x_ref, w_ref, out_ref):
    h = lax.dot_general(x_ref[...], w_ref[...], (((1,), (0,)), ((), ())),
                        preferred_element_type=_F32)
    out_ref[0] = h[:, :DSP]
    out_ref[1] = h[:, DSP:]


def _mm2_body(pa_ref, pb_ref, b1_ref, w2_ref, out_ref):
    h_lo = jnp.maximum(pa_ref[0] + pa_ref[1] + b1_ref[0:1, :], 0.0)
    h_hi = jnp.maximum(pb_ref[0] + pb_ref[1] + b1_ref[1:2, :], 0.0)
    w2 = w2_ref[...]
    out_ref[...] = (
        lax.dot_general(h_lo, w2[:DSP], (((1,), (0,)), ((), ())),
                        preferred_element_type=_F32)
        + lax.dot_general(h_hi, w2[DSP:], (((1,), (0,)), ((), ())),
                          preferred_element_type=_F32))


def _softmax(z):
    m = jnp.max(z, axis=1, keepdims=True)
    e = jnp.exp(z - m)
    return e / jnp.sum(e, axis=1, keepdims=True)


def _emb_body(q_ref, b2_ref, emb_ref, data_ref):
    emb = q_ref[0] + q_ref[1] + b2_ref[0:1, :]
    emb_ref[...] = emb
    nrm = jnp.sqrt(jnp.sum(emb * emb, axis=1, keepdims=True))
    data_ref[...] = emb / nrm


def _cluster_body(data_ref, mu0_ref, niter_ref, mu_ref):
    data = data_ref[...]

    def body(_, mu):
        dist = lax.dot_general(data, mu, (((1,), (1,)), ((), ())),
                               preferred_element_type=_F32)
        r = _softmax(CLUSTER_TEMP * dist)
        cr = jnp.sum(r, axis=0)
        cm = lax.dot_general(r, data, (((0,), (0,)), ((), ())),
                             preferred_element_type=_F32)
        return cm / cr[:, None]

    mu_ref[...] = lax.fori_loop(0, niter_ref[0, 0] + 1, body, mu0_ref[...])


def _assign_body(data_ref, mu_ref, r_ref, dist_ref):
    dist = lax.dot_general(data_ref[...], mu_ref[...], (((1,), (1,)), ((), ())),
                           preferred_element_type=_F32)
    dist_ref[...] = dist
    r_ref[...] = _softmax(CLUSTER_TEMP * dist)


def kernel(x, edge_index, edge_weight, W1, b1, W2, b2, init_mu, num_iter):
    n, nfeat = x.shape
    nout = W2.shape[1]
    k = init_mu.shape[0]
    e = edge_weight.shape[0]

    ept = ((e + NW * 256 - 1) // (NW * 256)) * 256
    ep = ept * NW
    nrow = ept // 128
    pad = ep - e
    dst = jnp.pad(edge_index[0], (0, pad))
    src = jnp.pad(edge_index[1], (0, pad))
    w = jnp.pad(edge_weight, (0, pad))
    src_t = src.reshape(NW, nrow, 128)
    dst_t = dst.reshape(NW, nrow, 128)
    ng = ept // 64
    w_t = jnp.broadcast_to(w[:, None], (ep, LANES)).reshape(NW, ng, 8, 128)

    n_pad = ((n + 8 * NS - 1) // (8 * NS)) * (8 * NS)
    spmm = _make_spmm(n_pad, nrow)

    def run_spmm(sup):
        out = spmm(sup, src_t, dst_t, w_t).reshape(2, n_pad, DSP)
        return out[:, :n]

    rb = 1000
    grid = n // rb
    sup1 = pl.pallas_call(
        _mm1_body,
        grid=(grid,),
        in_specs=[pl.BlockSpec((rb, nfeat), lambda i: (i, 0)),
                  pl.BlockSpec((nfeat, 2 * DSP), lambda i: (0, 0))],
        out_specs=pl.BlockSpec((2, rb, DSP), lambda i: (0, i, 0)),
        out_shape=jax.ShapeDtypeStruct((2, n, DSP), _F32),
    )(x, W1)

    pa = run_spmm(sup1[0])
    pb = run_spmm(sup1[1])

    sup2 = pl.pallas_call(
        _mm2_body,
        grid=(grid,),
        in_specs=[pl.BlockSpec((2, rb, DSP), lambda i: (0, i, 0)),
                  pl.BlockSpec((2, rb, DSP), lambda i: (0, i, 0)),
                  pl.BlockSpec((2, DSP), lambda i: (0, 0)),
                  pl.BlockSpec((nfeat, nout), lambda i: (0, 0))],
        out_specs=pl.BlockSpec((rb, nout), lambda i: (i, 0)),
        out_shape=jax.ShapeDtypeStruct((n, nout), _F32),
    )(pa, pb, b1.reshape(2, DSP), W2)

    q = run_spmm(sup2)

    embeds, data = pl.pallas_call(
        _emb_body,
        grid=(grid,),
        in_specs=[pl.BlockSpec((2, rb, DSP), lambda i: (0, i, 0)),
                  pl.BlockSpec((1, nout), lambda i: (0, 0))],
        out_specs=(pl.BlockSpec((rb, nout), lambda i: (i, 0)),
                   pl.BlockSpec((rb, nout), lambda i: (i, 0))),
        out_shape=(jax.ShapeDtypeStruct((n, nout), _F32),
                   jax.ShapeDtypeStruct((n, nout), _F32)),
    )(q, b2.reshape(1, nout))

    niter = jnp.asarray(num_iter, jnp.int32).reshape(1, 1)
    mu = pl.pallas_call(
        _cluster_body,
        in_specs=[pl.BlockSpec(memory_space=pltpu.VMEM),
                  pl.BlockSpec(memory_space=pltpu.VMEM),
                  pl.BlockSpec(memory_space=pltpu.SMEM)],
        out_shape=jax.ShapeDtypeStruct((k, nout), _F32),
    )(data, init_mu, niter)

    r, dist = pl.pallas_call(
        _assign_body,
        grid=(grid,),
        in_specs=[pl.BlockSpec((rb, nout), lambda i: (i, 0)),
                  pl.BlockSpec((k, nout), lambda i: (0, 0))],
        out_specs=(pl.BlockSpec((rb, k), lambda i: (i, 0)),
                   pl.BlockSpec((rb, k), lambda i: (i, 0))),
        out_shape=(jax.ShapeDtypeStruct((n, k), _F32),
                   jax.ShapeDtypeStruct((n, k), _F32)),
    )(data, mu)

    return (mu, r, embeds, dist)


# speedup vs baseline: 2.5994x; 1.0921x over previous
import functools

import jax
import jax.numpy as jnp
from jax import lax
from jax.experimental import pallas as pl
from jax.experimental.pallas import tpu as pltpu
from jax.experimental.pallas import tpu_sc as plsc

NC = 2
NS = 16
NW = NC * NS
LANES = 16
CHUNK = 32
DSP = 128
CLUSTER_TEMP = 30.0
_F32 = jnp.float32


NBUF = 4


def _spmm_body(n, nrow0, nrow1, sup_hbm, src_hbm, dst_hbm, w_hbm, out_hbm,
               acc, src_v, dst_v, w_vs, rows_vs,
               g0, g1, g2, g3, s0, s1, s2, s3, zsem):
    gsem = (g0, g1, g2, g3)
    ssem = (s0, s1, s2, s3)
    c = lax.axis_index("c")
    s = lax.axis_index("s")
    rpt = n // NS
    zrows = 8
    nz = rpt // zrows
    nrow_c = jnp.where(c == 0, nrow0, nrow1)
    base_row = c * NS * nrow0 + s * nrow_c
    base_ng = 2 * base_row
    ng = nrow_c * 2

    zero16 = jnp.zeros((LANES,), _F32)
    for i in range(zrows):
        for k in range(DSP // LANES):
            w_vs[0, i, pl.ds(k * LANES, LANES)] = zero16

    def zfire(i, _):
        pltpu.async_copy(w_vs.at[0], acc.at[pl.ds(s * rpt + i * zrows, zrows)], zsem)
        return 0
    lax.fori_loop(0, nz, zfire, 0)

    nrow_max = max(nrow0, nrow1)
    pltpu.sync_copy(src_hbm.at[pl.ds(base_row, nrow_max)], src_v)
    pltpu.sync_copy(dst_hbm.at[pl.ds(base_row, nrow_max)], dst_v)

    def zdrain(i, _):
        pltpu.make_async_copy(w_vs.at[0], acc.at[pl.ds(0, zrows)], zsem).wait()
        return 0
    lax.fori_loop(0, nz, zdrain, 0)

    for j0 in range(2):
        pltpu.async_copy(w_hbm.at[base_ng + j0], w_vs.at[j0], gsem[j0])
        pltpu.async_copy(sup_hbm.at[src_v.at[0, pl.ds(j0 * 64, 64)]],
                         rows_vs.at[j0], gsem[j0])

    plsc.subcore_barrier()

    def outer(g, _):
        for b in range(NBUF):
            j = g * NBUF + b
            jr = j // 2
            col0 = (b % 2) * 64
            pltpu.make_async_copy(sup_hbm.at[pl.ds(0, 64)], rows_vs.at[b],
                                  gsem[b]).wait()
            pltpu.make_async_copy(w_hbm.at[0], w_vs.at[b % 2], gsem[b]).wait()

            def escale(e2, _):
                w16 = w_vs[b % 2, e2 // 8, pl.ds((e2 % 8) * LANES, LANES)]
                for k in range(DSP // LANES):
                    sl = pl.ds(k * LANES, LANES)
                    rows_vs[b, e2, sl] = rows_vs[b, e2, sl] * w16
                return 0
            lax.fori_loop(0, 64, escale, 0)

            pltpu.async_copy(rows_vs.at[b],
                             acc.at[dst_v.at[jr, pl.ds(col0, 64)]],
                             ssem[b], add=True)

            jn = j + 2
            bn = (b + 2) % NBUF

            @pl.when(jn < ng)
            def _():
                @pl.when(j >= 2)
                def _():
                    pltpu.make_async_copy(rows_vs.at[bn], acc.at[pl.ds(0, 64)],
                                          ssem[bn]).wait()
                pltpu.async_copy(w_hbm.at[base_ng + jn], w_vs.at[bn % 2], gsem[bn])
                pltpu.async_copy(
                    sup_hbm.at[src_v.at[jn // 2, pl.ds(col0, 64)]],
                    rows_vs.at[bn], gsem[bn])
        return 0
    lax.fori_loop(0, nrow_c // 2, outer, 0)

    for b in range(NBUF):
        pltpu.make_async_copy(rows_vs.at[b], acc.at[pl.ds(0, 64)],
                              ssem[b]).wait()

    plsc.subcore_barrier()

    pltpu.sync_copy(acc.at[pl.ds(s * rpt, rpt)],
                    out_hbm.at[pl.ds(c * n + s * rpt, rpt)])


def _make_spmm(n, nrow0, nrow1):
    mesh = plsc.VectorSubcoreMesh(core_axis_name="c", subcore_axis_name="s",
                                  num_cores=NC, num_subcores=NS)
    nrow_max = max(nrow0, nrow1)
    return pl.kernel(
        functools.partial(_spmm_body, n, nrow0, nrow1),
        out_type=jax.ShapeDtypeStruct((NC * n, DSP), _F32),
        mesh=mesh,
        scratch_types=[
            pltpu.VMEM_SHARED((n, DSP), _F32),
            pltpu.VMEM((nrow_max, 128), jnp.int32),
            pltpu.VMEM((nrow_max, 128), jnp.int32),
            pltpu.VMEM((2, 8, 128), _F32),
            pltpu.VMEM((NBUF, 64, DSP), _F32),
        ] + [pltpu.SemaphoreType.DMA] * 9,
    )


def _mm1_body(x_ref, w_ref, out_ref):
    h = lax.dot_general(x_ref[...], w_ref[...], (((1,), (0,)), ((), ())),
                        preferred_element_type=_F32)
    out_ref[0] = h[:, :DSP]
    out_ref[1] = h[:, DSP:]


def _mm2_body(pa_ref, pb_ref, b1_ref, w2_ref, out_ref):
    h_lo = jnp.maximum(pa_ref[0] + pa_ref[1] + b1_ref[0:1, :], 0.0)
    h_hi = jnp.maximum(pb_ref[0] + pb_ref[1] + b1_ref[1:2, :], 0.0)
    w2 = w2_ref[...]
    out_ref[...] = (
        lax.dot_general(h_lo, w2[:DSP], (((1,), (0,)), ((), ())),
                        preferred_element_type=_F32)
        + lax.dot_general(h_hi, w2[DSP:], (((1,), (0,)), ((), ())),
                          preferred_element_type=_F32))


def _softmax(z):
    m = jnp.max(z, axis=1, keepdims=True)
    e = jnp.exp(z - m)
    return e / jnp.sum(e, axis=1, keepdims=True)


def _emb_body(q_ref, b2_ref, emb_ref, data_ref):
    emb = q_ref[0] + q_ref[1] + b2_ref[0:1, :]
    emb_ref[...] = emb
    nrm = jnp.sqrt(jnp.sum(emb * emb, axis=1, keepdims=True))
    data_ref[...] = emb / nrm


def _cluster_body(data_ref, mu0_ref, niter_ref, mu_ref):
    data = data_ref[...]

    def body(_, mu):
        dist = lax.dot_general(data, mu, (((1,), (1,)), ((), ())),
                               preferred_element_type=_F32)
        r = _softmax(CLUSTER_TEMP * dist)
        cr = jnp.sum(r, axis=0)
        cm = lax.dot_general(r, data, (((0,), (0,)), ((), ())),
                             preferred_element_type=_F32)
        return cm / cr[:, None]

    mu_ref[...] = lax.fori_loop(0, niter_ref[0, 0] + 1, body, mu0_ref[...])


def _assign_body(data_ref, mu_ref, r_ref, dist_ref):
    dist = lax.dot_general(data_ref[...], mu_ref[...], (((1,), (1,)), ((), ())),
                           preferred_element_type=_F32)
    dist_ref[...] = dist
    r_ref[...] = _softmax(CLUSTER_TEMP * dist)


def kernel(x, edge_index, edge_weight, W1, b1, W2, b2, init_mu, num_iter):
    n, nfeat = x.shape
    nout = W2.shape[1]
    k = init_mu.shape[0]
    e = edge_weight.shape[0]

    ept = ((e + NW * 256 - 1) // (NW * 256)) * 256
    ep = ept * NW
    rows_pair = 2 * ept // 128
    nrow1 = max(8, int(round(rows_pair / 3.6 / 8)) * 8)
    nrow0 = rows_pair - nrow1
    nrow_max = max(nrow0, nrow1)
    tot_rows = NS * rows_pair
    pad = ep - e
    dst = jnp.pad(edge_index[0], (0, pad))
    src = jnp.pad(edge_index[1], (0, pad))
    w = jnp.pad(edge_weight, (0, pad))
    tailpad = nrow_max - min(nrow0, nrow1)
    src_t = jnp.pad(src.reshape(tot_rows, 128), ((0, tailpad), (0, 0)))
    dst_t = jnp.pad(dst.reshape(tot_rows, 128), ((0, tailpad), (0, 0)))
    w_t = jnp.pad(jnp.broadcast_to(w[:, None], (ep, LANES)).reshape(2 * tot_rows, 8, 128),
                  ((0, 2 * tailpad), (0, 0), (0, 0)))

    n_pad = ((n + 8 * NS - 1) // (8 * NS)) * (8 * NS)
    spmm = _make_spmm(n_pad, nrow0, nrow1)

    def run_spmm(sup):
        out = spmm(sup, src_t, dst_t, w_t).reshape(2, n_pad, DSP)
        return out[:, :n]

    rb = 1000
    grid = n // rb
    sup1 = pl.pallas_call(
        _mm1_body,
        grid=(grid,),
        in_specs=[pl.BlockSpec((rb, nfeat), lambda i: (i, 0)),
                  pl.BlockSpec((nfeat, 2 * DSP), lambda i: (0, 0))],
        out_specs=pl.BlockSpec((2, rb, DSP), lambda i: (0, i, 0)),
        out_shape=jax.ShapeDtypeStruct((2, n, DSP), _F32),
    )(x, W1)

    pa = run_spmm(sup1[0])
    pb = run_spmm(sup1[1])

    sup2 = pl.pallas_call(
        _mm2_body,
        grid=(grid,),
        in_specs=[pl.BlockSpec((2, rb, DSP), lambda i: (0, i, 0)),
                  pl.BlockSpec((2, rb, DSP), lambda i: (0, i, 0)),
                  pl.BlockSpec((2, DSP), lambda i: (0, 0)),
                  pl.BlockSpec((nfeat, nout), lambda i: (0, 0))],
        out_specs=pl.BlockSpec((rb, nout), lambda i: (i, 0)),
        out_shape=jax.ShapeDtypeStruct((n, nout), _F32),
    )(pa, pb, b1.reshape(2, DSP), W2)

    q = run_spmm(sup2)

    embeds, data = pl.pallas_call(
        _emb_body,
        grid=(grid,),
        in_specs=[pl.BlockSpec((2, rb, DSP), lambda i: (0, i, 0)),
                  pl.BlockSpec((1, nout), lambda i: (0, 0))],
        out_specs=(pl.BlockSpec((rb, nout), lambda i: (i, 0)),
                   pl.BlockSpec((rb, nout), lambda i: (i, 0))),
        out_shape=(jax.ShapeDtypeStruct((n, nout), _F32),
                   jax.ShapeDtypeStruct((n, nout), _F32)),
    )(q, b2.reshape(1, nout))

    niter = jnp.asarray(num_iter, jnp.int32).reshape(1, 1)
    mu = pl.pallas_call(
        _cluster_body,
        in_specs=[pl.BlockSpec(memory_space=pltpu.VMEM),
                  pl.BlockSpec(memory_space=pltpu.VMEM),
                  pl.BlockSpec(memory_space=pltpu.SMEM)],
        out_shape=jax.ShapeDtypeStruct((k, nout), _F32),
    )(data, init_mu, niter)

    r, dist = pl.pallas_call(
        _assign_body,
        grid=(grid,),
        in_specs=[pl.BlockSpec((rb, nout), lambda i: (i, 0)),
                  pl.BlockSpec((k, nout), lambda i: (0, 0))],
        out_specs=(pl.BlockSpec((rb, k), lambda i: (i, 0)),
                   pl.BlockSpec((rb, k), lambda i: (i, 0))),
        out_shape=(jax.ShapeDtypeStruct((n, k), _F32),
                   jax.ShapeDtypeStruct((n, k), _F32)),
    )(data, mu)

    return (mu, r, embeds, dist)
